# baseline (speedup 1.0000x reference)
import functools

import jax
import jax.numpy as jnp
from jax import lax
from jax.experimental import pallas as pl
from jax.experimental.pallas import tpu as pltpu
from jax.experimental.pallas import tpu_sc as plsc

_B = 16384
_T = 24
_D = 128
_NC, _NS = 2, 16
_NW = _NC * _NS
_ROWS_PER_W = _B * _T // _NW
_REP = 32
_CHUNK = _REP * _T
_NCHUNK = _ROWS_PER_W // _CHUNK


def _sc_body(tab_hbm, out_hbm, rep_v, sem):
    wid = lax.axis_index("c") * _NS + lax.axis_index("s")
    base = wid * _ROWS_PER_W
    stage = [
        pltpu.async_copy(tab_hbm, rep_v.at[pl.ds(j * _T, _T)], sem)
        for j in range(_REP)
    ]
    for c in stage:
        c.wait()
    copies = [
        pltpu.async_copy(rep_v, out_hbm.at[pl.ds(base + j * _CHUNK, _CHUNK)], sem)
        for j in range(_NCHUNK)
    ]
    for c in copies:
        c.wait()


@functools.partial(jax.jit, static_argnums=())
def _sc_broadcast(table):
    k = pl.kernel(
        _sc_body,
        out_type=jax.ShapeDtypeStruct((_B * _T, _D), jnp.float32),
        scratch_types=[
            pltpu.VMEM((_CHUNK, _D), jnp.float32),
            pltpu.SemaphoreType.DMA,
        ],
        mesh=plsc.VectorSubcoreMesh(core_axis_name="c", subcore_axis_name="s"),
    )
    return k(table)


def kernel(t_future, batch_size, table):
    del t_future, batch_size
    return _sc_broadcast(table).reshape(_B, _T, _D)


# speedup vs baseline: 1.9981x; 1.9981x over previous
import jax
import jax.numpy as jnp
from jax.experimental import pallas as pl

_B = 16384
_BB = 1024


def _body(tab_ref, out_ref):
    out_ref[...] = jnp.broadcast_to(tab_ref[...][None], out_ref.shape)


def kernel(t_future, batch_size, table):
    del t_future, batch_size
    T, D = table.shape
    return pl.pallas_call(
        _body,
        grid=(_B // _BB,),
        in_specs=[pl.BlockSpec((T, D), lambda i: (0, 0))],
        out_specs=pl.BlockSpec((_BB, T, D), lambda i: (i, 0, 0)),
        out_shape=jax.ShapeDtypeStruct((_B, T, D), table.dtype),
    )(table)
